# baseline (speedup 1.0000x reference)
import jax
import jax.numpy as jnp
from jax.experimental import pallas as pl
from jax.experimental.pallas import tpu as pltpu

_B_BLK = 64
_LANES = 128


def _sim_kernel(f_ref, pt_ref, o_ref):
    f = f_ref[...]
    norm = jnp.sqrt(jnp.sum(f * f, axis=1, keepdims=True))
    fn = f / jnp.maximum(norm, 1e-12)
    o_ref[...] = jnp.dot(fn, pt_ref[...], preferred_element_type=jnp.float32)


def kernel(feats, prototypes):
    batch, emb = feats.shape
    n_classes = prototypes.shape[0]
    pad_n = ((n_classes + _LANES - 1) // _LANES) * _LANES
    pt = jnp.pad(prototypes.T, ((0, 0), (0, pad_n - n_classes)))
    out = pl.pallas_call(
        _sim_kernel,
        grid=(pl.cdiv(batch, _B_BLK),),
        in_specs=[
            pl.BlockSpec((_B_BLK, emb), lambda i: (i, 0)),
            pl.BlockSpec((emb, pad_n), lambda i: (0, 0)),
        ],
        out_specs=pl.BlockSpec((_B_BLK, pad_n), lambda i: (i, 0)),
        out_shape=jax.ShapeDtypeStruct((batch, pad_n), jnp.float32),
    )(feats, pt)
    return out[:, :n_classes]


# speedup vs baseline: 1.0054x; 1.0054x over previous
import jax
import jax.numpy as jnp
from jax.experimental import pallas as pl
from jax.experimental.pallas import tpu as pltpu

_B_BLK = 32
_LANES = 128


def _sim_kernel(f_ref, pt_ref, o_ref):
    f = f_ref[...]
    norm = jnp.sqrt(jnp.sum(f * f, axis=1, keepdims=True))
    fn = f / jnp.maximum(norm, 1e-12)
    o_ref[...] = jnp.dot(fn, pt_ref[...], preferred_element_type=jnp.float32)


def kernel(feats, prototypes):
    batch, emb = feats.shape
    n_classes = prototypes.shape[0]
    pad_n = ((n_classes + _LANES - 1) // _LANES) * _LANES
    pt = jnp.pad(prototypes.T, ((0, 0), (0, pad_n - n_classes)))
    out = pl.pallas_call(
        _sim_kernel,
        grid=(pl.cdiv(batch, _B_BLK),),
        in_specs=[
            pl.BlockSpec((_B_BLK, emb), lambda i: (i, 0)),
            pl.BlockSpec((emb, pad_n), lambda i: (0, 0)),
        ],
        out_specs=pl.BlockSpec((_B_BLK, pad_n), lambda i: (i, 0)),
        out_shape=jax.ShapeDtypeStruct((batch, pad_n), jnp.float32),
    )(feats, pt)
    return out[:, :n_classes]
